# Initial kernel scaffold; baseline (speedup 1.0000x reference)
import jax
import jax.numpy as jnp
from jax.experimental import pallas as pl


def kernel(hidden_states, gate_weight):
    raise NotImplementedError("write your pallas kernel here")


# speedup vs baseline: 1.0271x; 1.0271x over previous
import functools

import jax
import jax.numpy as jnp
from jax.experimental import pallas as pl
from jax.experimental.pallas import tpu as pltpu
from jax.experimental.pallas import tpu_sc as plsc

_NUM_EXPERTS = 64
_TOP_K = 8
_TM = 512


def _router_block(x_ref, w_ref, rw_ref, ti_ref, cnt_ref, before_ref):
    t = pl.program_id(0)
    logits = jax.lax.dot_general(
        x_ref[...], w_ref[...], (((1,), (1,)), ((), ())),
        preferred_element_type=jnp.float32,
        precision=jax.lax.Precision.DEFAULT,
    )
    m = jnp.max(logits, axis=1, keepdims=True)
    e = jnp.exp(logits - m)
    probs = e / jnp.sum(e, axis=1, keepdims=True)

    iota_e = jax.lax.broadcasted_iota(jnp.int32, (_TM, _NUM_EXPERTS), 1)
    iota_k = jax.lax.broadcasted_iota(jnp.int32, (_TM, _TOP_K), 1)
    work = probs
    rw = jnp.zeros((_TM, _TOP_K), jnp.float32)
    ti = jnp.zeros((_TM, _TOP_K), jnp.int32)
    ssum = jnp.zeros((_TM, 1), jnp.float32)
    onehot = jnp.zeros((_TM, _NUM_EXPERTS), jnp.int32)
    for k in range(_TOP_K):
        mk = jnp.max(work, axis=1, keepdims=True)
        am = jnp.min(
            jnp.where(work == mk, iota_e, _NUM_EXPERTS), axis=1, keepdims=True)
        rw = rw + jnp.where(iota_k == k, mk, 0.0)
        ti = ti + jnp.where(iota_k == k, am, 0)
        ssum = ssum + mk
        onehot = onehot + (iota_e == am).astype(jnp.int32)
        work = jnp.where(iota_e == am, -1.0, work)
    rw_ref[...] = rw / (ssum + 1e-8)
    ti_ref[...] = ti

    @pl.when(t == 0)
    def _():
        cnt_ref[...] = jnp.zeros((1, _NUM_EXPERTS), jnp.int32)

    before_ref[pl.ds(t, 1), :] = cnt_ref[...]
    cnt_ref[...] += jnp.sum(onehot, axis=0, keepdims=True)


def _router(x, w):
    tokens = x.shape[0]
    grid = tokens // _TM
    return pl.pallas_call(
        _router_block,
        grid=(grid,),
        in_specs=[
            pl.BlockSpec((_TM, x.shape[1]), lambda t: (t, 0)),
            pl.BlockSpec((_NUM_EXPERTS, x.shape[1]), lambda t: (0, 0)),
        ],
        out_specs=[
            pl.BlockSpec((_TM, _TOP_K), lambda t: (t, 0)),
            pl.BlockSpec((_TM, _TOP_K), lambda t: (t, 0)),
            pl.BlockSpec((1, _NUM_EXPERTS), lambda t: (0, 0)),
            pl.BlockSpec((grid, _NUM_EXPERTS), lambda t: (0, 0)),
        ],
        out_shape=[
            jax.ShapeDtypeStruct((tokens, _TOP_K), jnp.float32),
            jax.ShapeDtypeStruct((tokens, _TOP_K), jnp.int32),
            jax.ShapeDtypeStruct((1, _NUM_EXPERTS), jnp.int32),
            jax.ShapeDtypeStruct((grid, _NUM_EXPERTS), jnp.int32),
        ],
    )(x, w)


def _make_sc_dispatch(nslots):
    chunk = nslots // 32
    rows = chunk // 128
    mesh = plsc.VectorSubcoreMesh(core_axis_name="c", subcore_axis_name="s")

    @functools.partial(
        pl.kernel,
        out_type=(
            jax.ShapeDtypeStruct((_NUM_EXPERTS,), jnp.int32),
            jax.ShapeDtypeStruct((nslots,), jnp.int32),
        ),
        mesh=mesh,
        compiler_params=pltpu.CompilerParams(needs_layout_passes=False),
        scratch_types=[
            pltpu.VMEM((chunk,), jnp.int32),
            pltpu.VMEM((_NUM_EXPERTS,), jnp.int32),
            pltpu.VMEM((_NUM_EXPERTS,), jnp.int32),
            pltpu.VMEM((_NUM_EXPERTS,), jnp.int32),
            pltpu.VMEM((_NUM_EXPERTS,), jnp.int32),
            pltpu.VMEM((rows, 128), jnp.int32),
            pltpu.VMEM((rows, 128), jnp.int32),
            pltpu.SemaphoreType.DMA,
        ],
    )
    def dispatch(experts, counts, before, offs_out, gather_out,
                 ebuf, ctv, bfv, curs, offsv, posbuf, valbuf, sem):
        wid = jax.lax.axis_index("c") * 16 + jax.lax.axis_index("s")
        pltpu.sync_copy(experts.at[pl.ds(wid * chunk, chunk)], ebuf)
        pltpu.sync_copy(counts, ctv)
        pltpu.sync_copy(before.at[pl.ds(wid * _NUM_EXPERTS, _NUM_EXPERTS)], bfv)

        iota = jax.lax.iota(jnp.int32, 16)
        carry = jnp.int32(0)
        for j in range(_NUM_EXPERTS // 16):
            sl = pl.ds(j * 16, 16)
            v = ctv[sl]
            inc = plsc.cumsum(v)
            curs[sl] = (inc - v + carry) + bfv[sl]
            offsv[sl] = inc + carry
            carry = carry + jnp.sum(v)

        @pl.when(wid == 0)
        def _():
            pltpu.sync_copy(offsv, offs_out)

        base_gid = wid * chunk
        for row in range(rows):
            for c in range(8):
                sl = pl.ds(c * 16, 16)
                e = ebuf[pl.ds(row * 128 + c * 16, 16)]
                r1, last = plsc.scan_count(e)
                pos = plsc.load_gather(curs, [e]) + (r1 - 1)
                posbuf[row, sl] = pos
                valbuf[row, sl] = base_gid + row * 128 + c * 16 + iota
                plsc.addupdate_scatter(curs, [e], r1, mask=last)

        cps = [
            pltpu.async_copy(valbuf.at[row], gather_out.at[posbuf.at[row]], sem)
            for row in range(rows)
        ]
        for cp in cps:
            cp.wait()

    return dispatch


def kernel(hidden_states, gate_weight):
    b, s, h = hidden_states.shape
    x = hidden_states.reshape(b * s, h)
    rw, ti, cnts, before = _router(x, gate_weight)
    nslots = b * s * _TOP_K
    offs, gather = _make_sc_dispatch(nslots)(
        ti.reshape(-1), cnts.reshape(-1), before.reshape(-1))
    zero = jnp.float32(0.0)
    return (rw.reshape(-1), ti, offs, gather, zero, zero, zero)


# speedup vs baseline: 1.9204x; 1.8697x over previous
import functools

import jax
import jax.numpy as jnp
from jax.experimental import pallas as pl
from jax.experimental.pallas import tpu as pltpu
from jax.experimental.pallas import tpu_sc as plsc

_NUM_EXPERTS = 64
_TOP_K = 8
_TM = 512


def _router_block(x_ref, w_ref, rw_ref, ti_ref, cnt_ref, before_ref):
    t = pl.program_id(0)
    logits = jax.lax.dot_general(
        x_ref[...], w_ref[...], (((1,), (1,)), ((), ())),
        preferred_element_type=jnp.float32,
        precision=jax.lax.Precision.DEFAULT,
    )
    m = jnp.max(logits, axis=1, keepdims=True)
    e = jnp.exp(logits - m)
    probs = e / jnp.sum(e, axis=1, keepdims=True)

    iota_e = jax.lax.broadcasted_iota(jnp.int32, (_TM, _NUM_EXPERTS), 1)
    iota_k = jax.lax.broadcasted_iota(jnp.int32, (_TM, _TOP_K), 1)
    work = probs
    rw = jnp.zeros((_TM, _TOP_K), jnp.float32)
    ti = jnp.zeros((_TM, _TOP_K), jnp.int32)
    ssum = jnp.zeros((_TM, 1), jnp.float32)
    onehot = jnp.zeros((_TM, _NUM_EXPERTS), jnp.int32)
    for k in range(_TOP_K):
        mk = jnp.max(work, axis=1, keepdims=True)
        am = jnp.min(
            jnp.where(work == mk, iota_e, _NUM_EXPERTS), axis=1, keepdims=True)
        rw = rw + jnp.where(iota_k == k, mk, 0.0)
        ti = ti + jnp.where(iota_k == k, am, 0)
        ssum = ssum + mk
        onehot = onehot + (iota_e == am).astype(jnp.int32)
        work = jnp.where(iota_e == am, -1.0, work)
    rw_ref[...] = rw / (ssum + 1e-8)
    ti_ref[...] = ti

    @pl.when(t == 0)
    def _():
        cnt_ref[...] = jnp.zeros((1, _NUM_EXPERTS), jnp.int32)

    before_ref[pl.ds(t, 1), :] = cnt_ref[...]
    cnt_ref[...] += jnp.sum(onehot, axis=0, keepdims=True)


def _router(x, w):
    tokens = x.shape[0]
    grid = tokens // _TM
    return pl.pallas_call(
        _router_block,
        grid=(grid,),
        in_specs=[
            pl.BlockSpec((_TM, x.shape[1]), lambda t: (t, 0)),
            pl.BlockSpec((_NUM_EXPERTS, x.shape[1]), lambda t: (0, 0)),
        ],
        out_specs=[
            pl.BlockSpec((_TM, _TOP_K), lambda t: (t, 0)),
            pl.BlockSpec((_TM, _TOP_K), lambda t: (t, 0)),
            pl.BlockSpec((1, _NUM_EXPERTS), lambda t: (0, 0)),
            pl.BlockSpec((grid, _NUM_EXPERTS), lambda t: (0, 0)),
        ],
        out_shape=[
            jax.ShapeDtypeStruct((tokens, _TOP_K), jnp.float32),
            jax.ShapeDtypeStruct((tokens, _TOP_K), jnp.int32),
            jax.ShapeDtypeStruct((1, _NUM_EXPERTS), jnp.int32),
            jax.ShapeDtypeStruct((grid, _NUM_EXPERTS), jnp.int32),
        ],
    )(x, w)


def _make_sc_dispatch(nslots):
    chunk = nslots // 32
    rows = chunk // 128
    half = nslots // 2
    mesh = plsc.VectorSubcoreMesh(core_axis_name="c", subcore_axis_name="s")

    @functools.partial(
        pl.kernel,
        out_type=(
            jax.ShapeDtypeStruct((_NUM_EXPERTS,), jnp.int32),
            jax.ShapeDtypeStruct((nslots,), jnp.int32),
        ),
        mesh=mesh,
        compiler_params=pltpu.CompilerParams(needs_layout_passes=False),
        scratch_types=[
            pltpu.VMEM((2 * chunk,), jnp.int32),
            pltpu.VMEM((_NUM_EXPERTS,), jnp.int32),
            pltpu.VMEM((2 * _NUM_EXPERTS,), jnp.int32),
            pltpu.VMEM((2 * _NUM_EXPERTS,), jnp.int32),
            pltpu.VMEM((_NUM_EXPERTS,), jnp.int32),
            pltpu.VMEM((2 * rows, 128), jnp.int32),
            pltpu.VMEM((2 * rows, 128), jnp.int32),
            pltpu.VMEM_SHARED((nslots,), jnp.int32),
            pltpu.SemaphoreType.DMA,
        ],
    )
    def dispatch(experts, counts, before, offs_out, gather_out,
                 ebuf, ctv, bfv, curs, offsv, posbuf, valbuf, gshared, sem):
        cid = jax.lax.axis_index("c")
        sid = jax.lax.axis_index("s")
        pltpu.sync_copy(experts.at[pl.ds(sid * 2 * chunk, 2 * chunk)], ebuf)
        pltpu.sync_copy(counts, ctv)
        pltpu.sync_copy(
            before.at[pl.ds(sid * 2 * _NUM_EXPERTS, 2 * _NUM_EXPERTS)], bfv)

        iota = jax.lax.iota(jnp.int32, 16)
        carry = jnp.int32(0)
        incs = []
        for j in range(_NUM_EXPERTS // 16):
            sl = pl.ds(j * 16, 16)
            v = ctv[sl]
            inc = plsc.cumsum(v)
            excl = inc - v + carry
            curs[sl] = excl + bfv[sl]
            curs[pl.ds(_NUM_EXPERTS + j * 16, 16)] = (
                excl + bfv[pl.ds(_NUM_EXPERTS + j * 16, 16)])
            incs.append(inc + carry)
            carry = carry + jnp.sum(v)

        @pl.when((cid == 0) & (sid == 0))
        def _():
            for j in range(_NUM_EXPERTS // 16):
                offsv[pl.ds(j * 16, 16)] = incs[j]
            pltpu.sync_copy(offsv, offs_out)

        base_gid = sid * 2 * chunk
        for row in range(rows):
            for c in range(8):
                sl = pl.ds(c * 16, 16)
                for j in range(2):
                    src = j * chunk + row * 128 + c * 16
                    e = ebuf[pl.ds(src, 16)]
                    r1, last = plsc.scan_count(e)
                    ej = e + j * _NUM_EXPERTS
                    pos = plsc.load_gather(curs, [ej]) + (r1 - 1)
                    posbuf[j * rows + row, sl] = pos
                    valbuf[j * rows + row, sl] = base_gid + src + iota
                    plsc.addupdate_scatter(curs, [ej], r1, mask=last)

        cps = [
            pltpu.async_copy(valbuf.at[r], gshared.at[posbuf.at[r]], sem)
            for r in range(2 * rows)
        ]
        for cp in cps:
            cp.wait()
        plsc.subcore_barrier()

        out_lo = cid * half + sid * (half // 16)
        pltpu.sync_copy(gshared.at[pl.ds(out_lo, half // 16)],
                        gather_out.at[pl.ds(out_lo, half // 16)])

    return dispatch


def kernel(hidden_states, gate_weight):
    b, s, h = hidden_states.shape
    x = hidden_states.reshape(b * s, h)
    rw, ti, cnts, before = _router(x, gate_weight)
    nslots = b * s * _TOP_K
    offs, gather = _make_sc_dispatch(nslots)(
        ti.reshape(-1), cnts.reshape(-1), before.reshape(-1))
    zero = jnp.float32(0.0)
    return (rw.reshape(-1), ti, offs, gather, zero, zero, zero)


# speedup vs baseline: 2.0729x; 1.0794x over previous
import functools

import jax
import jax.numpy as jnp
from jax.experimental import pallas as pl
from jax.experimental.pallas import tpu as pltpu
from jax.experimental.pallas import tpu_sc as plsc

_NUM_EXPERTS = 64
_TOP_K = 8
_TM = 512


def _router_block(x_ref, w_ref, rw_ref, ti_ref, cnt_ref, before_ref):
    t = pl.program_id(0)
    logits = jax.lax.dot_general(
        x_ref[...], w_ref[...], (((1,), (1,)), ((), ())),
        preferred_element_type=jnp.float32,
        precision=jax.lax.Precision.DEFAULT,
    )
    m = jnp.max(logits, axis=1, keepdims=True)
    e = jnp.exp(logits - m)
    probs = e / jnp.sum(e, axis=1, keepdims=True)

    iota_e = jax.lax.broadcasted_iota(jnp.int32, (_TM, _NUM_EXPERTS), 1)
    iota_k = jax.lax.broadcasted_iota(jnp.int32, (_TM, _TOP_K), 1)
    work = probs
    rw = jnp.zeros((_TM, _TOP_K), jnp.float32)
    ti = jnp.zeros((_TM, _TOP_K), jnp.int32)
    ssum = jnp.zeros((_TM, 1), jnp.float32)
    for k in range(_TOP_K):
        mk = jnp.max(work, axis=1, keepdims=True)
        am = jnp.argmax(work, axis=1).astype(jnp.int32)[:, None]
        rw = rw + jnp.where(iota_k == k, mk, 0.0)
        ti = ti + jnp.where(iota_k == k, am, 0)
        ssum = ssum + mk
        work = jnp.where(iota_e == am, -1.0, work)
    rw_ref[...] = rw / (ssum + 1e-8)
    ti_ref[...] = ti

    @pl.when(t == 0)
    def _():
        cnt_ref[...] = jnp.zeros((1, _NUM_EXPERTS), jnp.int32)

    before_ref[pl.ds(t, 1), :] = cnt_ref[...]
    selected = (work == -1.0).astype(jnp.float32)
    ones_row = jnp.ones((1, _TM), jnp.float32)
    bcnt = jax.lax.dot_general(
        ones_row, selected, (((1,), (0,)), ((), ())),
        preferred_element_type=jnp.float32,
        precision=jax.lax.Precision.HIGHEST,
    )
    cnt_ref[...] += bcnt.astype(jnp.int32)


def _router(x, w):
    tokens = x.shape[0]
    grid = tokens // _TM
    return pl.pallas_call(
        _router_block,
        grid=(grid,),
        in_specs=[
            pl.BlockSpec((_TM, x.shape[1]), lambda t: (t, 0)),
            pl.BlockSpec((_NUM_EXPERTS, x.shape[1]), lambda t: (0, 0)),
        ],
        out_specs=[
            pl.BlockSpec((_TM, _TOP_K), lambda t: (t, 0)),
            pl.BlockSpec((_TM, _TOP_K), lambda t: (t, 0)),
            pl.BlockSpec((1, _NUM_EXPERTS), lambda t: (0, 0)),
            pl.BlockSpec((grid, _NUM_EXPERTS), lambda t: (0, 0)),
        ],
        out_shape=[
            jax.ShapeDtypeStruct((tokens, _TOP_K), jnp.float32),
            jax.ShapeDtypeStruct((tokens, _TOP_K), jnp.int32),
            jax.ShapeDtypeStruct((1, _NUM_EXPERTS), jnp.int32),
            jax.ShapeDtypeStruct((grid, _NUM_EXPERTS), jnp.int32),
        ],
    )(x, w)


def _make_sc_dispatch(nslots):
    chunk = nslots // 32
    rows = chunk // 128
    half = nslots // 2
    rpc = chunk // (_TM * _TOP_K)
    mesh = plsc.VectorSubcoreMesh(core_axis_name="c", subcore_axis_name="s")

    @functools.partial(
        pl.kernel,
        out_type=(
            jax.ShapeDtypeStruct((_NUM_EXPERTS,), jnp.int32),
            jax.ShapeDtypeStruct((nslots,), jnp.int32),
        ),
        mesh=mesh,
        compiler_params=pltpu.CompilerParams(needs_layout_passes=False),
        scratch_types=[
            pltpu.VMEM((2 * chunk,), jnp.int32),
            pltpu.VMEM((_NUM_EXPERTS,), jnp.int32),
            pltpu.VMEM((2 * rpc * _NUM_EXPERTS,), jnp.int32),
            pltpu.VMEM((2 * _NUM_EXPERTS,), jnp.int32),
            pltpu.VMEM((_NUM_EXPERTS,), jnp.int32),
            pltpu.VMEM((2 * rows, 128), jnp.int32),
            pltpu.VMEM((2 * rows, 128), jnp.int32),
            pltpu.VMEM_SHARED((nslots,), jnp.int32),
            pltpu.SemaphoreType.DMA,
        ],
    )
    def dispatch(experts, counts, before, offs_out, gather_out,
                 ebuf, ctv, bfv, curs, offsv, posbuf, valbuf, gshared, sem):
        cid = jax.lax.axis_index("c")
        sid = jax.lax.axis_index("s")
        pltpu.sync_copy(experts.at[pl.ds(sid * 2 * chunk, 2 * chunk)], ebuf)
        pltpu.sync_copy(counts, ctv)
        pltpu.sync_copy(
            before.at[pl.ds(sid * 2 * rpc * _NUM_EXPERTS,
                            2 * rpc * _NUM_EXPERTS)], bfv)

        iota = jax.lax.iota(jnp.int32, 16)
        carry = jnp.int32(0)
        incs = []
        for j in range(_NUM_EXPERTS // 16):
            sl = pl.ds(j * 16, 16)
            v = ctv[sl]
            inc = plsc.cumsum(v)
            excl = inc - v + carry
            curs[sl] = excl + bfv[sl]
            curs[pl.ds(_NUM_EXPERTS + j * 16, 16)] = (
                excl + bfv[pl.ds(rpc * _NUM_EXPERTS + j * 16, 16)])
            incs.append(inc + carry)
            carry = carry + jnp.sum(v)

        @pl.when((cid == 0) & (sid == 0))
        def _():
            for j in range(_NUM_EXPERTS // 16):
                offsv[pl.ds(j * 16, 16)] = incs[j]
            pltpu.sync_copy(offsv, offs_out)

        base_gid = sid * 2 * chunk
        for row in range(rows):
            for c in range(8):
                sl = pl.ds(c * 16, 16)
                for j in range(2):
                    src = j * chunk + row * 128 + c * 16
                    e = ebuf[pl.ds(src, 16)]
                    r1, last = plsc.scan_count(e)
                    ej = e + j * _NUM_EXPERTS
                    pos = plsc.load_gather(curs, [ej]) + (r1 - 1)
                    posbuf[j * rows + row, sl] = pos
                    valbuf[j * rows + row, sl] = base_gid + src + iota
                    plsc.addupdate_scatter(curs, [ej], r1, mask=last)

        cps = [
            pltpu.async_copy(valbuf.at[r], gshared.at[posbuf.at[r]], sem)
            for r in range(2 * rows)
        ]
        for cp in cps:
            cp.wait()
        plsc.subcore_barrier()

        out_lo = cid * half + sid * (half // 16)
        pltpu.sync_copy(gshared.at[pl.ds(out_lo, half // 16)],
                        gather_out.at[pl.ds(out_lo, half // 16)])

    return dispatch


def kernel(hidden_states, gate_weight):
    b, s, h = hidden_states.shape
    x = hidden_states.reshape(b * s, h)
    rw, ti, cnts, before = _router(x, gate_weight)
    nslots = b * s * _TOP_K
    offs, gather = _make_sc_dispatch(nslots)(
        ti.reshape(-1), cnts.reshape(-1), before.reshape(-1))
    zero = jnp.float32(0.0)
    return (rw.reshape(-1), ti, offs, gather, zero, zero, zero)


# speedup vs baseline: 2.0952x; 1.0107x over previous
import functools

import jax
import jax.numpy as jnp
from jax.experimental import pallas as pl
from jax.experimental.pallas import tpu as pltpu
from jax.experimental.pallas import tpu_sc as plsc

_NUM_EXPERTS = 64
_TOP_K = 8
_TM = 512


def _router_block(x_ref, w_ref, rw_ref, ti_ref, cnt_ref, before_ref):
    t = pl.program_id(0)
    logits = jax.lax.dot_general(
        x_ref[...], w_ref[...], (((1,), (1,)), ((), ())),
        preferred_element_type=jnp.float32,
        precision=jax.lax.Precision.DEFAULT,
    )
    m = jnp.max(logits, axis=1, keepdims=True)
    e = jnp.exp(logits - m)
    probs = e / jnp.sum(e, axis=1, keepdims=True)

    iota_e = jax.lax.broadcasted_iota(jnp.int32, (_TM, _NUM_EXPERTS), 1)
    iota_k = jax.lax.broadcasted_iota(jnp.int32, (_TM, _TOP_K), 1)
    work = probs
    rw = jnp.zeros((_TM, _TOP_K), jnp.float32)
    ti = jnp.zeros((_TM, _TOP_K), jnp.int32)
    ssum = jnp.zeros((_TM, 1), jnp.float32)
    for k in range(_TOP_K):
        mk = jnp.max(work, axis=1, keepdims=True)
        am = jnp.argmax(work, axis=1).astype(jnp.int32)[:, None]
        rw = rw + jnp.where(iota_k == k, mk, 0.0)
        ti = ti + jnp.where(iota_k == k, am, 0)
        ssum = ssum + mk
        work = jnp.where(iota_e == am, -1.0, work)
    rw_ref[...] = rw / (ssum + 1e-8)
    ti_ref[...] = ti

    @pl.when(t == 0)
    def _():
        cnt_ref[...] = jnp.zeros((1, _NUM_EXPERTS), jnp.int32)

    before_ref[pl.ds(t, 1), :] = cnt_ref[...]
    selected = (work == -1.0).astype(jnp.float32)
    ones_row = jnp.ones((1, _TM), jnp.float32)
    bcnt = jax.lax.dot_general(
        ones_row, selected, (((1,), (0,)), ((), ())),
        preferred_element_type=jnp.float32,
        precision=jax.lax.Precision.HIGHEST,
    )
    cnt_ref[...] += bcnt.astype(jnp.int32)


def _router(x, w):
    tokens = x.shape[0]
    grid = tokens // _TM
    return pl.pallas_call(
        _router_block,
        grid=(grid,),
        in_specs=[
            pl.BlockSpec((_TM, x.shape[1]), lambda t: (t, 0)),
            pl.BlockSpec((_NUM_EXPERTS, x.shape[1]), lambda t: (0, 0)),
        ],
        out_specs=[
            pl.BlockSpec((_TM, _TOP_K), lambda t: (t, 0)),
            pl.BlockSpec((_TM, _TOP_K), lambda t: (t, 0)),
            pl.BlockSpec((1, _NUM_EXPERTS), lambda t: (0, 0)),
            pl.BlockSpec((grid, _NUM_EXPERTS), lambda t: (0, 0)),
        ],
        out_shape=[
            jax.ShapeDtypeStruct((tokens, _TOP_K), jnp.float32),
            jax.ShapeDtypeStruct((tokens, _TOP_K), jnp.int32),
            jax.ShapeDtypeStruct((1, _NUM_EXPERTS), jnp.int32),
            jax.ShapeDtypeStruct((grid, _NUM_EXPERTS), jnp.int32),
        ],
    )(x, w)


def _make_sc_dispatch(nslots):
    chunk = nslots // 32
    rows = chunk // 128
    half = nslots // 2
    rpc = chunk // (_TM * _TOP_K)
    mesh = plsc.VectorSubcoreMesh(core_axis_name="c", subcore_axis_name="s")

    @functools.partial(
        pl.kernel,
        out_type=(
            jax.ShapeDtypeStruct((_NUM_EXPERTS,), jnp.int32),
            jax.ShapeDtypeStruct((nslots,), jnp.int32),
        ),
        mesh=mesh,
        compiler_params=pltpu.CompilerParams(needs_layout_passes=False),
        scratch_types=[
            pltpu.VMEM((2 * chunk,), jnp.int32),
            pltpu.VMEM((_NUM_EXPERTS,), jnp.int32),
            pltpu.VMEM((2 * rpc * _NUM_EXPERTS,), jnp.int32),
            pltpu.VMEM((2 * _NUM_EXPERTS,), jnp.int32),
            pltpu.VMEM((_NUM_EXPERTS,), jnp.int32),
            pltpu.VMEM((2 * rows, 128), jnp.int32),
            pltpu.VMEM((2 * rows, 128), jnp.int32),
            pltpu.VMEM_SHARED((nslots,), jnp.int32),
            pltpu.SemaphoreType.DMA,
        ],
    )
    def dispatch(experts, counts, before, offs_out, gather_out,
                 ebuf, ctv, bfv, curs, offsv, posbuf, valbuf, gshared, sem):
        cid = jax.lax.axis_index("c")
        sid = jax.lax.axis_index("s")
        pltpu.sync_copy(experts.at[pl.ds(sid * 2 * chunk, 2 * chunk)], ebuf)
        pltpu.sync_copy(counts, ctv)
        pltpu.sync_copy(
            before.at[pl.ds(sid * 2 * rpc * _NUM_EXPERTS,
                            2 * rpc * _NUM_EXPERTS)], bfv)

        iota = jax.lax.iota(jnp.int32, 16)
        carry = jnp.int32(0)
        incs = []
        for j in range(_NUM_EXPERTS // 16):
            sl = pl.ds(j * 16, 16)
            v = ctv[sl]
            inc = plsc.cumsum(v)
            excl = inc - v + carry
            curs[sl] = excl + bfv[sl]
            curs[pl.ds(_NUM_EXPERTS + j * 16, 16)] = (
                excl + bfv[pl.ds(rpc * _NUM_EXPERTS + j * 16, 16)])
            incs.append(inc + carry)
            carry = carry + jnp.sum(v)

        @pl.when((cid == 0) & (sid == 0))
        def _():
            for j in range(_NUM_EXPERTS // 16):
                offsv[pl.ds(j * 16, 16)] = incs[j]
            pltpu.sync_copy(offsv, offs_out)

        base_gid = sid * 2 * chunk
        cps = []
        for row in range(rows):
            for c in range(8):
                sl = pl.ds(c * 16, 16)
                for j in range(2):
                    src = j * chunk + row * 128 + c * 16
                    e = ebuf[pl.ds(src, 16)]
                    r1, last = plsc.scan_count(e)
                    ej = e + j * _NUM_EXPERTS
                    pos = plsc.load_gather(curs, [ej]) + (r1 - 1)
                    posbuf[j * rows + row, sl] = pos
                    valbuf[j * rows + row, sl] = base_gid + src + iota
                    plsc.addupdate_scatter(curs, [ej], r1, mask=last)
            for j in range(2):
                r = j * rows + row
                cps.append(
                    pltpu.async_copy(valbuf.at[r], gshared.at[posbuf.at[r]], sem))
        for cp in cps:
            cp.wait()
        plsc.subcore_barrier()

        out_lo = cid * half + sid * (half // 16)
        pltpu.sync_copy(gshared.at[pl.ds(out_lo, half // 16)],
                        gather_out.at[pl.ds(out_lo, half // 16)])

    return dispatch


def kernel(hidden_states, gate_weight):
    b, s, h = hidden_states.shape
    x = hidden_states.reshape(b * s, h)
    rw, ti, cnts, before = _router(x, gate_weight)
    nslots = b * s * _TOP_K
    offs, gather = _make_sc_dispatch(nslots)(
        ti.reshape(-1), cnts.reshape(-1), before.reshape(-1))
    zero = jnp.float32(0.0)
    return (rw.reshape(-1), ti, offs, gather, zero, zero, zero)
